# baseline (speedup 1.0000x reference)
import functools
import math

import jax
import jax.numpy as jnp
from jax import lax
from jax.experimental import pallas as pl
from jax.experimental.pallas import tpu as pltpu
from jax.experimental.pallas import tpu_sc as plsc

D_MODEL = 64
SCALE = math.sqrt(D_MODEL)

_info = plsc.get_sparse_core_info()
NC, NS, L = _info.num_cores, _info.num_subcores, _info.num_lanes
NW = NC * NS

CHUNK = 128
NBUF = 4


def _make_kernel(B, D):
    assert B % NW == 0
    b_per_w = B // NW
    assert b_per_w % (CHUNK * NBUF) == 0
    n_chunks = b_per_w // CHUNK

    mesh = plsc.VectorSubcoreMesh(core_axis_name="c", subcore_axis_name="s")

    @functools.partial(
        pl.kernel,
        mesh=mesh,
        out_type=jax.ShapeDtypeStruct((B, D), jnp.float32),
        scratch_types=[
            pltpu.VMEM((n_chunks, CHUNK), jnp.int32),
            [pltpu.VMEM((CHUNK, D), jnp.float32) for _ in range(NBUF)],
            [pltpu.VMEM((CHUNK, D), jnp.float32) for _ in range(NBUF)],
            [pltpu.SemaphoreType.DMA for _ in range(NBUF)],
            [pltpu.SemaphoreType.DMA for _ in range(NBUF)],
        ],
        compiler_params=pltpu.CompilerParams(use_tc_tiling_on_sc=False),
    )
    def k(x_hbm, table_hbm, out_hbm, idx_v, gbufs, wbufs, gsems, wsems):
        wid = lax.axis_index("s") * NC + lax.axis_index("c")
        base = wid * b_per_w
        pltpu.sync_copy(x_hbm.at[wid], idx_v)

        def start_gather(g, b):
            pltpu.async_copy(table_hbm.at[idx_v.at[g]], gbufs[b], gsems[b])

        for b in range(NBUF):
            start_gather(b, b)

        def outer(kk, _):
            for b in range(NBUF):
                g = kk * NBUF + b
                pltpu.make_async_copy(table_hbm.at[idx_v.at[g]], gbufs[b],
                                      gsems[b]).wait()

                @pl.when(kk > 0)
                def _():
                    pltpu.make_async_copy(
                        wbufs[b],
                        out_hbm.at[pl.ds(base + (g - NBUF) * CHUNK, CHUNK)],
                        wsems[b]).wait()

                def scale_row(r, _):
                    for j in range(D // L):
                        sl = pl.ds(j * L, L)
                        wbufs[b][r, sl] = gbufs[b][r, sl] * SCALE
                    return 0

                lax.fori_loop(0, CHUNK, scale_row, 0, unroll=4)

                @pl.when(g + NBUF < n_chunks)
                def _():
                    start_gather(g + NBUF, b)

                pltpu.async_copy(wbufs[b],
                                 out_hbm.at[pl.ds(base + g * CHUNK, CHUNK)],
                                 wsems[b])
            return 0

        lax.fori_loop(0, n_chunks // NBUF, outer, 0)

        for b in range(NBUF):
            g = n_chunks - NBUF + b
            pltpu.make_async_copy(wbufs[b],
                                  out_hbm.at[pl.ds(base + g * CHUNK, CHUNK)],
                                  wsems[b]).wait()

    return k


def kernel(x, table):
    B = x.shape[0] * x.shape[1]
    D = table.shape[1]
    x3 = x.reshape(NW, (B // NW) // CHUNK, CHUNK).astype(jnp.int32)
    out = _make_kernel(B, D)(x3, table)
    return out.reshape(x.shape[0], x.shape[1], D)


# speedup vs baseline: 1.8837x; 1.8837x over previous
import functools
import math

import jax
import jax.numpy as jnp
from jax import lax
from jax.experimental import pallas as pl
from jax.experimental.pallas import tpu as pltpu
from jax.experimental.pallas import tpu_sc as plsc

D_MODEL = 64
SCALE = math.sqrt(D_MODEL)

_info = plsc.get_sparse_core_info()
NC, NS, L = _info.num_cores, _info.num_subcores, _info.num_lanes
NW = NC * NS

BBLK = 128
NK = BBLK // L
GBUF = 4
WBUF = 2


def _make_kernel(S, NB, D):
    CB = D // 8

    mesh = plsc.VectorSubcoreMesh(core_axis_name="c", subcore_axis_name="s")

    @functools.partial(
        pl.kernel,
        mesh=mesh,
        out_type=jax.ShapeDtypeStruct((S, CB, NB, 8, BBLK), jnp.float32),
        scratch_types=[
            pltpu.VMEM((S, BBLK), jnp.int32),
            [pltpu.VMEM((BBLK,), jnp.int32) for _ in range(GBUF)],
            [pltpu.VMEM((BBLK, 2 * D), jnp.float32) for _ in range(GBUF)],
            [pltpu.VMEM((CB, 8, BBLK), jnp.float32) for _ in range(WBUF)],
            [pltpu.SemaphoreType.DMA for _ in range(GBUF)],
            [pltpu.SemaphoreType.DMA for _ in range(WBUF)],
        ],
        compiler_params=pltpu.CompilerParams(
            use_tc_tiling_on_sc=True, needs_layout_passes=False),
    )
    def k(xt_hbm, t2_hbm, out_hbm, xbuf, gidx, gbufs, tbufs, gsems, wsems):
        w = lax.axis_index("s") * NC + lax.axis_index("c")

        pltpu.sync_copy(xt_hbm.at[:, pl.ds(w * BBLK, BBLK)], xbuf)

        def fill_gidx(s, g):
            for j in range(NK):
                sl = pl.ds(j * L, L)
                gidx[g][sl] = xbuf[s, sl] >> 1

        def start_gather(s, g):
            pltpu.async_copy(t2_hbm.at[gidx[g]], gbufs[g], gsems[g])

        for g in range(GBUF):
            fill_gidx(g, g)
            start_gather(g, g)

        iota = lax.iota(jnp.int32, L)
        mvecs = [(iota + d) & (L - 1) for d in range(L)]

        def step(s, g, p):
            pltpu.make_async_copy(t2_hbm.at[gidx[g]], gbufs[g],
                                  gsems[g]).wait()

            @pl.when(s >= WBUF)
            def _():
                pltpu.make_async_copy(tbufs[p], out_hbm.at[s - WBUF, :, w],
                                      wsems[p]).wait()

            def blk(jj):
                c0 = (jj >> 3) * L
                kk = jj & (NK - 1)
                rvec = iota + kk * L
                ovec = (xbuf[s, pl.ds(kk * L, L)] & 1) * D
                for d in range(L):
                    cvec = mvecs[d] + c0
                    vals = plsc.load_gather(gbufs[g], [rvec, ovec + cvec])
                    plsc.store_scatter(
                        tbufs[p], [cvec >> 3, cvec & 7, rvec], vals * SCALE)

            plsc.parallel_loop(0, (D // L) * NK, 1, unroll=2,
                               carry=None)(blk)

            @pl.when(s + GBUF < S)
            def _():
                fill_gidx(s + GBUF, g)
                start_gather(s + GBUF, g)

            pltpu.async_copy(tbufs[p], out_hbm.at[s, :, w], wsems[p])

        def quad(q, _):
            for i in range(GBUF):
                s = q * GBUF + i
                step(s, i, i % WBUF)
            return 0

        lax.fori_loop(0, S // GBUF, quad, 0)
        for p in range(WBUF):
            pltpu.make_async_copy(tbufs[p], out_hbm.at[S - WBUF + p, :, w],
                                  wsems[p]).wait()

    return k


def kernel(x, table):
    BT, S = x.shape
    V, D = table.shape
    NB = BT // BBLK
    t2 = table.reshape(V // 2, 2 * D)
    xt = x.astype(jnp.int32).T
    out5 = _make_kernel(S, NB, D)(xt, t2)
    return out5.transpose(2, 4, 0, 1, 3).reshape(BT, S, D)
